# baseline (speedup 1.0000x reference)
import functools

import jax
import jax.numpy as jnp
from jax import lax
from jax.experimental import pallas as pl
from jax.experimental.pallas import tpu as pltpu
from jax.experimental.pallas import tpu_sc as plsc

B = 16384
D = 64
W = 2 * D
NC = 2
NS = 16
NW = NC * NS
BPW = B // NW
CHUNK = 128
NCHUNK = BPW // CHUNK
PASS_ROWS = 256
NPASS = BPW // PASS_ROWS


def _sc_gather(uix2d, iix2d, ut2, it2):
    mesh = plsc.VectorSubcoreMesh(core_axis_name="c", subcore_axis_name="s")

    @functools.partial(
        pl.kernel,
        mesh=mesh,
        out_type=(
            jax.ShapeDtypeStruct((B, W), jnp.float32),
            jax.ShapeDtypeStruct((B, W), jnp.float32),
        ),
        scratch_types=[
            pltpu.VMEM((NCHUNK, CHUNK), jnp.int32),
            pltpu.VMEM((NCHUNK, CHUNK), jnp.int32),
            pltpu.VMEM((PASS_ROWS, W), jnp.float32),
            pltpu.VMEM((PASS_ROWS, W), jnp.float32),
            pltpu.SemaphoreType.DMA,
            pltpu.SemaphoreType.DMA,
        ],
    )
    def k(u_hbm, i_hbm, ut_hbm, it_hbm, qo_hbm, co_hbm,
          uix_v, iix_v, q_v, c_v, sem_q, sem_c):
        wid = lax.axis_index("s") * NC + lax.axis_index("c")
        base = wid * BPW
        pltpu.sync_copy(u_hbm.at[pl.ds(wid * NCHUNK, NCHUNK)], uix_v)
        pltpu.sync_copy(i_hbm.at[pl.ds(wid * NCHUNK, NCHUNK)], iix_v)
        for p in range(NPASS):
            copies = []
            for g in range(PASS_ROWS // CHUNK):
                gg = p * (PASS_ROWS // CHUNK) + g
                copies.append(pltpu.async_copy(
                    ut_hbm.at[uix_v.at[gg]],
                    q_v.at[pl.ds(g * CHUNK, CHUNK)], sem_q))
                copies.append(pltpu.async_copy(
                    it_hbm.at[iix_v.at[gg]],
                    c_v.at[pl.ds(g * CHUNK, CHUNK)], sem_c))
            for cp in copies:
                cp.wait()
            pltpu.sync_copy(q_v, qo_hbm.at[pl.ds(base + p * PASS_ROWS, PASS_ROWS)])
            pltpu.sync_copy(c_v, co_hbm.at[pl.ds(base + p * PASS_ROWS, PASS_ROWS)])

    return k(uix2d, iix2d, ut2, it2)


def _tc_cosine(q2, c2, user, item):
    def body(q_ref, c_ref, u_ref, i_ref, o_ref):
        qfull = q_ref[...]
        cfull = c_ref[...]
        upar = (u_ref[...] & 1)[:, None]
        ipar = (i_ref[...] & 1)[:, None]
        qv = jnp.where(upar == 0, qfull[:, :D], qfull[:, D:])
        cv = jnp.where(ipar == 0, cfull[:, :D], cfull[:, D:])
        eps = jnp.float32(1e-8)
        dot = jnp.sum(qv * cv, axis=-1)
        qn = jnp.maximum(jnp.sqrt(jnp.sum(qv * qv, axis=-1)), eps)
        cn = jnp.maximum(jnp.sqrt(jnp.sum(cv * cv, axis=-1)), eps)
        o_ref[...] = dot / (qn * cn)

    return pl.pallas_call(
        body,
        out_shape=jax.ShapeDtypeStruct((B,), jnp.float32),
    )(q2, c2, user, item)


def kernel(user, item, user_table, item_table):
    ut2 = user_table.reshape(user_table.shape[0] // 2, W)
    it2 = item_table.reshape(item_table.shape[0] // 2, W)
    uix2d = (user >> 1).reshape(NW * NCHUNK, CHUNK)
    iix2d = (item >> 1).reshape(NW * NCHUNK, CHUNK)
    q2, c2 = _sc_gather(uix2d, iix2d, ut2, it2)
    return _tc_cosine(q2, c2, user, item)


# speedup vs baseline: 2.5294x; 2.5294x over previous
import functools

import jax
import jax.numpy as jnp
from jax import lax
from jax.experimental import pallas as pl
from jax.experimental.pallas import tpu as pltpu
from jax.experimental.pallas import tpu_sc as plsc

B = 16384
D = 64
HALF = 499712
W = 4096
NBLK = 123
YROWS = NBLK * W
NC = 2
NS = 16
NW = NC * NS
BPW = B // NW
CHUNK = 128
NCHUNK = BPW // CHUNK


def _tc_convert(tT):
    def body(a_ref, b_ref, o_ref):
        o_ref[...] = jnp.concatenate([a_ref[...], b_ref[...]], axis=0).T

    return pl.pallas_call(
        body,
        grid=(NBLK,),
        in_specs=[
            pl.BlockSpec((D, W), lambda j: (0, j)),
            pl.BlockSpec((D, W), lambda j: (0, NBLK - 1 + j)),
        ],
        out_specs=pl.BlockSpec((W, 2 * D), lambda j: (j, 0)),
        out_shape=jax.ShapeDtypeStruct((YROWS, 2 * D), jnp.float32),
        compiler_params=pltpu.CompilerParams(
            dimension_semantics=("parallel",)),
    )(tT, tT)


def _sc_gather(idx2d, table2):
    mesh = plsc.VectorSubcoreMesh(core_axis_name="c", subcore_axis_name="s")

    @functools.partial(
        pl.kernel,
        mesh=mesh,
        out_type=jax.ShapeDtypeStruct((B, 2 * D), jnp.float32),
        scratch_types=[
            pltpu.VMEM((NCHUNK, CHUNK), jnp.int32),
            pltpu.VMEM((BPW, 2 * D), jnp.float32),
            pltpu.SemaphoreType.DMA,
        ],
    )
    def k(ix_hbm, t_hbm, o_hbm, ix_v, rows_v, sem):
        wid = lax.axis_index("s") * NC + lax.axis_index("c")
        base = wid * BPW
        pltpu.sync_copy(ix_hbm.at[pl.ds(wid * NCHUNK, NCHUNK)], ix_v)
        copies = []
        for g in range(NCHUNK):
            copies.append(pltpu.async_copy(
                t_hbm.at[ix_v.at[g]],
                rows_v.at[pl.ds(g * CHUNK, CHUNK)], sem))
        for cp in copies:
            cp.wait()
        pltpu.sync_copy(rows_v, o_hbm.at[pl.ds(base, BPW)])

    return k(idx2d, table2)


def _tc_cosine(qg, cg, user, item):
    def body(q_ref, c_ref, u_ref, i_ref, o_ref):
        qfull = q_ref[...]
        cfull = c_ref[...]
        uhi = u_ref[...][:, None]
        ihi = i_ref[...][:, None]
        qv = jnp.where(uhi >= HALF, qfull[:, D:], qfull[:, :D])
        cv = jnp.where(ihi >= HALF, cfull[:, D:], cfull[:, :D])
        eps = jnp.float32(1e-8)
        dot = jnp.sum(qv * cv, axis=-1)
        qn = jnp.maximum(jnp.sqrt(jnp.sum(qv * qv, axis=-1)), eps)
        cn = jnp.maximum(jnp.sqrt(jnp.sum(cv * cv, axis=-1)), eps)
        o_ref[...] = dot / (qn * cn)

    return pl.pallas_call(
        body,
        out_shape=jax.ShapeDtypeStruct((B,), jnp.float32),
    )(qg, cg, user, item)


def kernel(user, item, user_table, item_table):
    yu = _tc_convert(user_table.T)
    yi = _tc_convert(item_table.T)
    uix = jnp.where(user >= HALF, user - HALF, user).reshape(NW * NCHUNK, CHUNK)
    iix = jnp.where(item >= HALF, item - HALF, item).reshape(NW * NCHUNK, CHUNK)
    qg = _sc_gather(uix, yu)
    cg = _sc_gather(iix, yi)
    return _tc_cosine(qg, cg, user, item)


# speedup vs baseline: 2.8602x; 1.1308x over previous
import functools

import jax
import jax.numpy as jnp
from jax import lax
from jax.experimental import pallas as pl
from jax.experimental.pallas import tpu as pltpu
from jax.experimental.pallas import tpu_sc as plsc

B = 16384
D = 64
HALF = 499712
W = 8192
NBLK = 62
YROWS = NBLK * W
NC = 2
NS = 16
NW = NC * NS
BPW = B // NW
CHUNK = 128
NCHUNK = BPW // CHUNK


def _tc_convert(tT):
    def body(a_ref, b_ref, o_ref):
        o_ref[...] = jnp.concatenate([a_ref[...], b_ref[...]], axis=0).T

    return pl.pallas_call(
        body,
        grid=(NBLK,),
        in_specs=[
            pl.BlockSpec((D, W), lambda j: (0, j)),
            pl.BlockSpec((D, W), lambda j: (0, NBLK - 1 + j)),
        ],
        out_specs=pl.BlockSpec((W, 2 * D), lambda j: (j, 0)),
        out_shape=jax.ShapeDtypeStruct((YROWS, 2 * D), jnp.float32),
        compiler_params=pltpu.CompilerParams(
            dimension_semantics=("parallel",)),
    )(tT, tT)


def _sc_gather(idx2d, table2):
    mesh = plsc.VectorSubcoreMesh(core_axis_name="c", subcore_axis_name="s")

    @functools.partial(
        pl.kernel,
        mesh=mesh,
        out_type=jax.ShapeDtypeStruct((B, 2 * D), jnp.float32),
        scratch_types=[
            pltpu.VMEM((NCHUNK, CHUNK), jnp.int32),
            pltpu.VMEM((BPW, 2 * D), jnp.float32),
            pltpu.SemaphoreType.DMA,
        ],
    )
    def k(ix_hbm, t_hbm, o_hbm, ix_v, rows_v, sem):
        wid = lax.axis_index("s") * NC + lax.axis_index("c")
        base = wid * BPW
        pltpu.sync_copy(ix_hbm.at[pl.ds(wid * NCHUNK, NCHUNK)], ix_v)
        copies = []
        for g in range(NCHUNK):
            copies.append(pltpu.async_copy(
                t_hbm.at[ix_v.at[g]],
                rows_v.at[pl.ds(g * CHUNK, CHUNK)], sem))
        for cp in copies:
            cp.wait()
        pltpu.sync_copy(rows_v, o_hbm.at[pl.ds(base, BPW)])

    return k(idx2d, table2)


def _tc_cosine(qg, cg, user, item):
    def body(q_ref, c_ref, u_ref, i_ref, o_ref):
        qfull = q_ref[...]
        cfull = c_ref[...]
        uhi = u_ref[...][:, None]
        ihi = i_ref[...][:, None]
        qv = jnp.where(uhi >= HALF, qfull[:, D:], qfull[:, :D])
        cv = jnp.where(ihi >= HALF, cfull[:, D:], cfull[:, :D])
        eps = jnp.float32(1e-8)
        dot = jnp.sum(qv * cv, axis=-1)
        qn = jnp.maximum(jnp.sqrt(jnp.sum(qv * qv, axis=-1)), eps)
        cn = jnp.maximum(jnp.sqrt(jnp.sum(cv * cv, axis=-1)), eps)
        o_ref[...] = dot / (qn * cn)

    return pl.pallas_call(
        body,
        out_shape=jax.ShapeDtypeStruct((B,), jnp.float32),
    )(qg, cg, user, item)


def kernel(user, item, user_table, item_table):
    yu = _tc_convert(user_table.T)
    yi = _tc_convert(item_table.T)
    uix = jnp.where(user >= HALF, user - HALF, user).reshape(NW * NCHUNK, CHUNK)
    iix = jnp.where(item >= HALF, item - HALF, item).reshape(NW * NCHUNK, CHUNK)
    qg = _sc_gather(uix, yu)
    cg = _sc_gather(iix, yi)
    return _tc_cosine(qg, cg, user, item)


# speedup vs baseline: 3.3530x; 1.1723x over previous
import functools

import jax
import jax.numpy as jnp
from jax import lax
from jax.experimental import pallas as pl
from jax.experimental.pallas import tpu as pltpu
from jax.experimental.pallas import tpu_sc as plsc

B = 16384
D = 64
W = 8192
OF = 245760
NBLK = 33
YROWS = NBLK * W
NC = 2
NS = 16
NW = NC * NS
BPW = B // NW
CHUNK = 128
NCHUNK = BPW // CHUNK


def _bf16_hi_bits(x):
    r = lax.bitcast_convert_type(x, jnp.int32)
    r = r + jnp.int32(0x7FFF) + (lax.shift_right_logical(r, 16) & 1)
    return lax.shift_right_logical(r, 16)


def _tc_convert(tT):
    def body(a_ref, b_ref, c_ref, d_ref, o_ref):
        lo = jnp.concatenate([a_ref[...], c_ref[...]], axis=0).T
        hi = jnp.concatenate([b_ref[...], d_ref[...]], axis=0).T
        o_ref[...] = _bf16_hi_bits(lo) | lax.shift_left(_bf16_hi_bits(hi), 16)

    return pl.pallas_call(
        body,
        grid=(NBLK,),
        in_specs=[
            pl.BlockSpec((D, W), lambda j: (0, j)),
            pl.BlockSpec((D, W), lambda j: (0, 30 + j)),
            pl.BlockSpec((D, W), lambda j: (0, 60 + j)),
            pl.BlockSpec((D, W), lambda j: (0, 90 + j)),
        ],
        out_specs=pl.BlockSpec((W, 2 * D), lambda j: (j, 0)),
        out_shape=jax.ShapeDtypeStruct((YROWS, 2 * D), jnp.int32),
        compiler_params=pltpu.CompilerParams(
            dimension_semantics=("parallel",)),
    )(tT, tT, tT, tT)


def _sc_gather(idx2d, table2):
    mesh = plsc.VectorSubcoreMesh(core_axis_name="c", subcore_axis_name="s")

    @functools.partial(
        pl.kernel,
        mesh=mesh,
        out_type=jax.ShapeDtypeStruct((B, 2 * D), jnp.int32),
        scratch_types=[
            pltpu.VMEM((NCHUNK, CHUNK), jnp.int32),
            pltpu.VMEM((BPW, 2 * D), jnp.int32),
            pltpu.SemaphoreType.DMA,
        ],
    )
    def k(ix_hbm, t_hbm, o_hbm, ix_v, rows_v, sem):
        wid = lax.axis_index("s") * NC + lax.axis_index("c")
        base = wid * BPW
        pltpu.sync_copy(ix_hbm.at[pl.ds(wid * NCHUNK, NCHUNK)], ix_v)
        copies = []
        for g in range(NCHUNK):
            copies.append(pltpu.async_copy(
                t_hbm.at[ix_v.at[g]],
                rows_v.at[pl.ds(g * CHUNK, CHUNK)], sem))
        for cp in copies:
            cp.wait()
        pltpu.sync_copy(rows_v, o_hbm.at[pl.ds(base, BPW)])

    return k(idx2d, table2)


def _unpack(x_ref, idx_ref):
    k = jnp.minimum(idx_ref[...] // OF, 3)[:, None]
    x = x_ref[...]
    s = jnp.where(k >= 2, x[:, D:], x[:, :D])
    bits = jnp.where((k & 1) == 1,
                     lax.shift_right_logical(s, 16) & jnp.int32(0xFFFF),
                     s & jnp.int32(0xFFFF))
    return lax.bitcast_convert_type(lax.shift_left(bits, 16), jnp.float32)


def _tc_cosine(qg, cg, user, item):
    def body(q_ref, c_ref, u_ref, i_ref, o_ref):
        qv = _unpack(q_ref, u_ref)
        cv = _unpack(c_ref, i_ref)
        eps = jnp.float32(1e-8)
        dot = jnp.sum(qv * cv, axis=-1)
        qn = jnp.maximum(jnp.sqrt(jnp.sum(qv * qv, axis=-1)), eps)
        cn = jnp.maximum(jnp.sqrt(jnp.sum(cv * cv, axis=-1)), eps)
        o_ref[...] = dot / (qn * cn)

    return pl.pallas_call(
        body,
        out_shape=jax.ShapeDtypeStruct((B,), jnp.float32),
    )(qg, cg, user, item)


def kernel(user, item, user_table, item_table):
    yu = _tc_convert(user_table.T)
    yi = _tc_convert(item_table.T)
    uq = jnp.minimum(user // OF, 3)
    iq = jnp.minimum(item // OF, 3)
    uix = (user - uq * OF).reshape(NW * NCHUNK, CHUNK)
    iix = (item - iq * OF).reshape(NW * NCHUNK, CHUNK)
    qg = _sc_gather(uix, yu)
    cg = _sc_gather(iix, yi)
    return _tc_cosine(qg, cg, user, item)
